# baseline (speedup 1.0000x reference)
import jax
import jax.numpy as jnp
from jax.experimental import pallas as pl
from jax.experimental.pallas import tpu as pltpu


def _dot(a, b):
    return jax.lax.dot(a.astype(jnp.bfloat16), b.astype(jnp.bfloat16),
                       preferred_element_type=jnp.float32)


def _body(x1_ref, x2_ref, wa0_ref, wb0_ref, wc0_ref, wd0_ref, we0_ref,
          wa1_ref, wb1_ref, wc1_ref, wd1_ref, we1_ref,
          ws_ref, bs_ref, wid_ref, bid_ref,
          y_ref, fp_ref, fc_ref, p0_ref, p1_ref, p2_ref, p3_ref, c_ref,
          w0s, w1a, w1b, fps, fcs):
    b = pl.program_id(0)
    t = pl.program_id(1)
    nb = pl.num_programs(0)
    nt = pl.num_programs(1)
    first = jnp.logical_and(b == 0, t == 0)

    @pl.when(first)
    def _zero_acc():
        fps[:, :] = jnp.zeros_like(fps)
        fcs[:, :] = jnp.zeros_like(fcs)

    @pl.when(first)
    def _stage_weights():
        w0s[:, 0:128] = wa0_ref[...].astype(jnp.bfloat16)
        w0s[:, 128:256] = (wb0_ref[...] * 0.5).astype(jnp.bfloat16)
        w0s[:, 256:384] = (wc0_ref[...] * 0.5).astype(jnp.bfloat16)
        w0s[:, 384:512] = (wd0_ref[...] * 0.5).astype(jnp.bfloat16)
        w1a[:, :] = jnp.zeros_like(w1a)
        w1a[0:128, 0:32] = wa1_ref[...].astype(jnp.bfloat16)
        w1a[0:128, 32:64] = (wb1_ref[...] * 0.5).astype(jnp.bfloat16)
        w1a[0:128, 64:96] = (wd1_ref[...] * 0.5).astype(jnp.bfloat16)
        w1a[0:128, 96:128] = (wc1_ref[...] * 0.5).astype(jnp.bfloat16)
        w1a[128:256, 96:128] = we1_ref[...].astype(jnp.bfloat16)
        w1b[:, :] = jnp.zeros_like(w1b)
        w1b[0:128, 0:32] = (wb1_ref[...] * 0.5).astype(jnp.bfloat16)
        w1b[0:128, 32:64] = wa1_ref[...].astype(jnp.bfloat16)
        w1b[0:128, 64:96] = (wc1_ref[...] * 0.5).astype(jnp.bfloat16)
        w1b[0:128, 96:128] = (wd1_ref[...] * 0.5).astype(jnp.bfloat16)
        w1b[128:256, 64:96] = we1_ref[...].astype(jnp.bfloat16)

    xa = x1_ref[0].astype(jnp.bfloat16)
    xb = x2_ref[0].astype(jnp.bfloat16)
    we0 = we0_ref[...]

    pa = jax.lax.dot(xa, w0s[...], preferred_element_type=jnp.float32)
    pb = jax.lax.dot(xb, w0s[...], preferred_element_type=jnp.float32)
    we0h = we0 * 0.5
    eh_ab = pa[:, 256:384] + pb[:, 384:512] + we0h
    eh_ba = pb[:, 256:384] + pa[:, 384:512] + we0h
    t_ab = jnp.tanh(eh_ab)
    t_ba = jnp.tanh(eh_ba)
    pbh = pb[:, 128:256]
    pah = pa[:, 128:256]
    ha = jax.nn.relu(pa[:, 0:128] + pbh + t_ba * pbh)
    hb = jax.nn.relu(pb[:, 0:128] + pah + t_ab * pah)
    hab = ha.astype(jnp.bfloat16)
    hbb = hb.astype(jnp.bfloat16)

    za = jnp.concatenate([hab, eh_ab.astype(jnp.bfloat16)], axis=1)
    zb = jnp.concatenate([hbb, eh_ba.astype(jnp.bfloat16)], axis=1)
    qa = jax.lax.dot(za, w1a[...], preferred_element_type=jnp.float32)
    qb = jax.lax.dot(zb, w1b[...], preferred_element_type=jnp.float32)
    t2 = jnp.tanh(qa + qb)
    tr = jnp.roll(t2, -64, axis=1)
    lane = jax.lax.broadcasted_iota(jnp.int32, qa.shape, 1)
    m0 = lane < 32
    gp = jnp.where(m0, qa, qb)
    gq = jnp.where(m0, qb, qa)
    g2 = jax.nn.relu(gp + gq + tr * gq)

    t_rows = xa.shape[0]
    inv = jnp.float32(1.0 / (t_rows * nt))
    ones8 = jnp.ones((8, t_rows), jnp.bfloat16)
    sa1 = jax.lax.dot(ones8, hab, preferred_element_type=jnp.float32)
    sb1 = jax.lax.dot(ones8, hbb, preferred_element_type=jnp.float32)
    s2 = jax.lax.dot(ones8, g2.astype(jnp.bfloat16),
                     preferred_element_type=jnp.float32)
    nb_rows = fps.shape[0]
    sel1 = jax.lax.broadcasted_iota(jnp.int32, (nb_rows, 128), 0) == b
    sel2 = jax.lax.broadcasted_iota(jnp.int32, (nb_rows, 32), 0) == b
    z1 = jnp.zeros((1, 128), jnp.float32)
    z2 = jnp.zeros((1, 32), jnp.float32)
    fps[:, 0:128] += jnp.where(sel1, sa1[0:1] * inv, z1)
    fps[:, 128:160] += jnp.where(sel2, s2[0:1, 0:32] * inv, z2)
    fcs[:, 0:128] += jnp.where(sel1, sb1[0:1] * inv, z1)
    fcs[:, 128:160] += jnp.where(sel2, s2[0:1, 32:64] * inv, z2)

    @pl.when(jnp.logical_and(b == nb - 1, t == nt - 1))
    def _head():
        fp = fps[...]
        fc = fcs[...]
        fp_ref[...] = fp
        fc_ref[...] = fc
        c_ref[...] = 0.5 * (fp + fc)
        y_ref[...] = _dot(jnp.abs(fp - fc), ws_ref[...]) + bs_ref[...]
        f = jnp.concatenate([fp, fc], axis=0)
        feat = fp.shape[1]
        nj = feat // 4
        k = jax.lax.broadcasted_iota(jnp.int32, (feat, feat), 0)
        c = jax.lax.broadcasted_iota(jnp.int32, (feat, feat), 1)
        sel = (k == 4 * (c % nj) + c // nj).astype(jnp.bfloat16)
        g = jax.lax.dot(f.astype(jnp.bfloat16), sel,
                        preferred_element_type=jnp.float32)
        for i, p_ref in enumerate((p0_ref, p1_ref, p2_ref, p3_ref)):
            p_ref[...] = _dot(g[:, nj * i:nj * (i + 1)], wid_ref[i]) \
                + bid_ref[i][None, :]


def kernel(x1_batch, x2_batch, WA0, WB0, WC0, WD0, WE0,
           WA1, WB1, WC1, WD1, WE1, Ws, bs, Wid, bid):
    B, L, d = x1_batch.shape
    d1 = WA1.shape[1]
    feat = d + d1
    out_dim = Ws.shape[1]

    nt = 2
    tl = L // nt
    y, fp, fc, p0, p1, p2, p3, center = pl.pallas_call(
        _body,
        grid=(B, nt),
        in_specs=[
            pl.BlockSpec((1, tl, d), lambda b, t: (b, t, 0)),
            pl.BlockSpec((1, tl, d), lambda b, t: (b, t, 0)),
            pl.BlockSpec((d, d), lambda b, t: (0, 0)),
            pl.BlockSpec((d, d), lambda b, t: (0, 0)),
            pl.BlockSpec((d, d), lambda b, t: (0, 0)),
            pl.BlockSpec((d, d), lambda b, t: (0, 0)),
            pl.BlockSpec((1, d), lambda b, t: (0, 0)),
            pl.BlockSpec((d, d1), lambda b, t: (0, 0)),
            pl.BlockSpec((d, d1), lambda b, t: (0, 0)),
            pl.BlockSpec((d, d1), lambda b, t: (0, 0)),
            pl.BlockSpec((d, d1), lambda b, t: (0, 0)),
            pl.BlockSpec((d, d1), lambda b, t: (0, 0)),
            pl.BlockSpec((feat, out_dim), lambda b, t: (0, 0)),
            pl.BlockSpec((1, out_dim), lambda b, t: (0, 0)),
            pl.BlockSpec((4, feat // 4, out_dim), lambda b, t: (0, 0, 0)),
            pl.BlockSpec((4, out_dim), lambda b, t: (0, 0)),
        ],
        out_specs=[
            pl.BlockSpec((B, out_dim), lambda b, t: (0, 0)),
            pl.BlockSpec((B, feat), lambda b, t: (0, 0)),
            pl.BlockSpec((B, feat), lambda b, t: (0, 0)),
            pl.BlockSpec((2 * B, out_dim), lambda b, t: (0, 0)),
            pl.BlockSpec((2 * B, out_dim), lambda b, t: (0, 0)),
            pl.BlockSpec((2 * B, out_dim), lambda b, t: (0, 0)),
            pl.BlockSpec((2 * B, out_dim), lambda b, t: (0, 0)),
            pl.BlockSpec((B, feat), lambda b, t: (0, 0)),
        ],
        out_shape=[
            jax.ShapeDtypeStruct((B, out_dim), jnp.float32),
            jax.ShapeDtypeStruct((B, feat), jnp.float32),
            jax.ShapeDtypeStruct((B, feat), jnp.float32),
            jax.ShapeDtypeStruct((2 * B, out_dim), jnp.float32),
            jax.ShapeDtypeStruct((2 * B, out_dim), jnp.float32),
            jax.ShapeDtypeStruct((2 * B, out_dim), jnp.float32),
            jax.ShapeDtypeStruct((2 * B, out_dim), jnp.float32),
            jax.ShapeDtypeStruct((B, feat), jnp.float32),
        ],
        scratch_shapes=[
            pltpu.VMEM((d, 4 * d), jnp.bfloat16),
            pltpu.VMEM((2 * d, d), jnp.bfloat16),
            pltpu.VMEM((2 * d, d), jnp.bfloat16),
            pltpu.VMEM((B, feat), jnp.float32),
            pltpu.VMEM((B, feat), jnp.float32),
        ],
    )(x1_batch, x2_batch, WA0, WB0, WC0, WD0, WE0,
      WA1, WB1, WC1, WD1, WE1, Ws, bs.reshape(1, -1), Wid, bid)

    return (y, fp, fc, p0, p1, p2, p3, center)


# speedup vs baseline: 1.2070x; 1.2070x over previous
import jax
import jax.numpy as jnp
from jax.experimental import pallas as pl
from jax.experimental.pallas import tpu as pltpu


def _dot(a, b):
    return jax.lax.dot(a.astype(jnp.bfloat16), b.astype(jnp.bfloat16),
                       preferred_element_type=jnp.float32)


def _body(x1_ref, x2_ref, wa0_ref, wb0_ref, wc0_ref, wd0_ref, we0_ref,
          wa1_ref, wb1_ref, wc1_ref, wd1_ref, we1_ref,
          ws_ref, bs_ref, wid_ref, bid_ref,
          y_ref, fp_ref, fc_ref, p0_ref, p1_ref, p2_ref, p3_ref, c_ref,
          w0s, w1a, w1b, fps, fcs):
    b = pl.program_id(0)
    nb = pl.num_programs(0)

    @pl.when(b == 0)
    def _stage_weights():
        w0s[:, 0:128] = wa0_ref[...].astype(jnp.bfloat16)
        w0s[:, 128:256] = (wb0_ref[...] * 0.5).astype(jnp.bfloat16)
        w0s[:, 256:384] = (wc0_ref[...] * 0.5).astype(jnp.bfloat16)
        w0s[:, 384:512] = (wd0_ref[...] * 0.5).astype(jnp.bfloat16)
        w1a[:, :] = jnp.zeros_like(w1a)
        w1a[0:128, 0:32] = wa1_ref[...].astype(jnp.bfloat16)
        w1a[0:128, 32:64] = (wb1_ref[...] * 0.5).astype(jnp.bfloat16)
        w1a[0:128, 64:96] = (wd1_ref[...] * 0.5).astype(jnp.bfloat16)
        w1a[0:128, 96:128] = (wc1_ref[...] * 0.5).astype(jnp.bfloat16)
        w1a[128:256, 96:128] = we1_ref[...].astype(jnp.bfloat16)
        w1b[:, :] = jnp.zeros_like(w1b)
        w1b[0:128, 0:32] = (wb1_ref[...] * 0.5).astype(jnp.bfloat16)
        w1b[0:128, 32:64] = wa1_ref[...].astype(jnp.bfloat16)
        w1b[0:128, 64:96] = (wc1_ref[...] * 0.5).astype(jnp.bfloat16)
        w1b[0:128, 96:128] = (wd1_ref[...] * 0.5).astype(jnp.bfloat16)
        w1b[128:256, 64:96] = we1_ref[...].astype(jnp.bfloat16)

    nbat, tl, dd = x1_ref.shape
    xa = x1_ref[...].reshape(nbat * tl, dd).astype(jnp.bfloat16)
    xb = x2_ref[...].reshape(nbat * tl, dd).astype(jnp.bfloat16)
    we0 = we0_ref[...]

    pa = jax.lax.dot(xa, w0s[...], preferred_element_type=jnp.float32)
    pb = jax.lax.dot(xb, w0s[...], preferred_element_type=jnp.float32)
    we0h = we0 * 0.5
    eh_ab = pa[:, 256:384] + pb[:, 384:512] + we0h
    eh_ba = pb[:, 256:384] + pa[:, 384:512] + we0h
    t_ab = jnp.tanh(eh_ab)
    t_ba = jnp.tanh(eh_ba)
    pbh = pb[:, 128:256]
    pah = pa[:, 128:256]
    ha = jax.nn.relu(pa[:, 0:128] + pbh + t_ba * pbh)
    hb = jax.nn.relu(pb[:, 0:128] + pah + t_ab * pah)
    hab = ha.astype(jnp.bfloat16)
    hbb = hb.astype(jnp.bfloat16)

    za = jnp.concatenate([hab, eh_ab.astype(jnp.bfloat16)], axis=1)
    zb = jnp.concatenate([hbb, eh_ba.astype(jnp.bfloat16)], axis=1)
    qa = jax.lax.dot(za, w1a[...], preferred_element_type=jnp.float32)
    qb = jax.lax.dot(zb, w1b[...], preferred_element_type=jnp.float32)
    t2 = jnp.tanh(qa + qb)
    tr = jnp.roll(t2, -64, axis=1)
    lane = jax.lax.broadcasted_iota(jnp.int32, qa.shape, 1)
    m0 = lane < 32
    gp = jnp.where(m0, qa, qb)
    gq = jnp.where(m0, qb, qa)
    g2 = jax.nn.relu(gp + gq + tr * gq)

    t_rows = xa.shape[0]
    inv = jnp.float32(1.0 / tl)
    orow = jax.lax.broadcasted_iota(jnp.int32, (8 * nbat, t_rows), 0)
    ocol = jax.lax.broadcasted_iota(jnp.int32, (8 * nbat, t_rows), 1)
    onesd = ((orow // 8) == (ocol // tl)).astype(jnp.bfloat16)
    sa1 = jax.lax.dot(onesd, hab, preferred_element_type=jnp.float32)
    sb1 = jax.lax.dot(onesd, hbb, preferred_element_type=jnp.float32)
    s2 = jax.lax.dot(onesd, g2.astype(jnp.bfloat16),
                     preferred_element_type=jnp.float32)
    nb_rows = fps.shape[0]
    for k in range(nbat):
        sel1 = jax.lax.broadcasted_iota(jnp.int32, (nb_rows, 128), 0) == nbat * b + k
        sel2 = jax.lax.broadcasted_iota(jnp.int32, (nb_rows, 32), 0) == nbat * b + k
        r = 8 * k
        fps[:, 0:128] = jnp.where(sel1, sa1[r:r + 1] * inv, fps[:, 0:128])
        fps[:, 128:160] = jnp.where(sel2, s2[r:r + 1, 0:32] * inv, fps[:, 128:160])
        fcs[:, 0:128] = jnp.where(sel1, sb1[r:r + 1] * inv, fcs[:, 0:128])
        fcs[:, 128:160] = jnp.where(sel2, s2[r:r + 1, 32:64] * inv, fcs[:, 128:160])

    @pl.when(b == nb - 1)
    def _head():
        fp = fps[...]
        fc = fcs[...]
        fp_ref[...] = fp
        fc_ref[...] = fc
        c_ref[...] = 0.5 * (fp + fc)
        y_ref[...] = _dot(jnp.abs(fp - fc), ws_ref[...]) + bs_ref[...]
        f = jnp.concatenate([fp, fc], axis=0)
        feat = fp.shape[1]
        nj = feat // 4
        k = jax.lax.broadcasted_iota(jnp.int32, (feat, feat), 0)
        c = jax.lax.broadcasted_iota(jnp.int32, (feat, feat), 1)
        sel = (k == 4 * (c % nj) + c // nj).astype(jnp.bfloat16)
        g = jax.lax.dot(f.astype(jnp.bfloat16), sel,
                        preferred_element_type=jnp.float32)
        for i, p_ref in enumerate((p0_ref, p1_ref, p2_ref, p3_ref)):
            p_ref[...] = _dot(g[:, nj * i:nj * (i + 1)], wid_ref[i]) \
                + bid_ref[i][None, :]


def kernel(x1_batch, x2_batch, WA0, WB0, WC0, WD0, WE0,
           WA1, WB1, WC1, WD1, WE1, Ws, bs, Wid, bid):
    B, L, d = x1_batch.shape
    d1 = WA1.shape[1]
    feat = d + d1
    out_dim = Ws.shape[1]

    nbat = 2
    y, fp, fc, p0, p1, p2, p3, center = pl.pallas_call(
        _body,
        grid=(B // nbat,),
        in_specs=[
            pl.BlockSpec((nbat, L, d), lambda b: (b, 0, 0)),
            pl.BlockSpec((nbat, L, d), lambda b: (b, 0, 0)),
            pl.BlockSpec((d, d), lambda b: (0, 0)),
            pl.BlockSpec((d, d), lambda b: (0, 0)),
            pl.BlockSpec((d, d), lambda b: (0, 0)),
            pl.BlockSpec((d, d), lambda b: (0, 0)),
            pl.BlockSpec((1, d), lambda b: (0, 0)),
            pl.BlockSpec((d, d1), lambda b: (0, 0)),
            pl.BlockSpec((d, d1), lambda b: (0, 0)),
            pl.BlockSpec((d, d1), lambda b: (0, 0)),
            pl.BlockSpec((d, d1), lambda b: (0, 0)),
            pl.BlockSpec((d, d1), lambda b: (0, 0)),
            pl.BlockSpec((feat, out_dim), lambda b: (0, 0)),
            pl.BlockSpec((1, out_dim), lambda b: (0, 0)),
            pl.BlockSpec((4, feat // 4, out_dim), lambda b: (0, 0, 0)),
            pl.BlockSpec((4, out_dim), lambda b: (0, 0)),
        ],
        out_specs=[
            pl.BlockSpec((B, out_dim), lambda b: (0, 0)),
            pl.BlockSpec((B, feat), lambda b: (0, 0)),
            pl.BlockSpec((B, feat), lambda b: (0, 0)),
            pl.BlockSpec((2 * B, out_dim), lambda b: (0, 0)),
            pl.BlockSpec((2 * B, out_dim), lambda b: (0, 0)),
            pl.BlockSpec((2 * B, out_dim), lambda b: (0, 0)),
            pl.BlockSpec((2 * B, out_dim), lambda b: (0, 0)),
            pl.BlockSpec((B, feat), lambda b: (0, 0)),
        ],
        out_shape=[
            jax.ShapeDtypeStruct((B, out_dim), jnp.float32),
            jax.ShapeDtypeStruct((B, feat), jnp.float32),
            jax.ShapeDtypeStruct((B, feat), jnp.float32),
            jax.ShapeDtypeStruct((2 * B, out_dim), jnp.float32),
            jax.ShapeDtypeStruct((2 * B, out_dim), jnp.float32),
            jax.ShapeDtypeStruct((2 * B, out_dim), jnp.float32),
            jax.ShapeDtypeStruct((2 * B, out_dim), jnp.float32),
            jax.ShapeDtypeStruct((B, feat), jnp.float32),
        ],
        scratch_shapes=[
            pltpu.VMEM((d, 4 * d), jnp.bfloat16),
            pltpu.VMEM((2 * d, d), jnp.bfloat16),
            pltpu.VMEM((2 * d, d), jnp.bfloat16),
            pltpu.VMEM((B, feat), jnp.float32),
            pltpu.VMEM((B, feat), jnp.float32),
        ],
    )(x1_batch, x2_batch, WA0, WB0, WC0, WD0, WE0,
      WA1, WB1, WC1, WD1, WE1, Ws, bs.reshape(1, -1), Wid, bid)

    return (y, fp, fc, p0, p1, p2, p3, center)
